# Initial kernel scaffold; baseline (speedup 1.0000x reference)
import jax
import jax.numpy as jnp
from jax.experimental import pallas as pl


def kernel(target_word, context_word, negative, u_weight, v_weight):
    raise NotImplementedError("write your pallas kernel here")


# speedup vs baseline: 1.7392x; 1.7392x over previous
import functools

import jax
import jax.numpy as jnp
from jax import lax
from jax.experimental import pallas as pl
from jax.experimental.pallas import tpu as pltpu
from jax.experimental.pallas import tpu_sc as plsc

_VOCAB = 1_000_000
_DIM = 64
_BATCH = 16384
_N_NEG = 5

_NC, _NS, _L = 2, 16, 16
_NW = _NC * _NS
_BPW = _BATCH // _NW
_CHUNK = 128
_NCHUNK = _BPW // _CHUNK
_KD = _DIM // _L


def _sc_dots_body(t_hbm, c_hbm, negt_hbm, u_hbm, v_hbm,
                  pos_hbm, neg_hbm,
                  t_idx, c_idx, n_idx, u_rows, c_rows, n_rows,
                  pos_v, neg_v, scr_p, scr_n, sem):
    wid = lax.axis_index("s") * _NC + lax.axis_index("c")
    iota16 = lax.iota(jnp.int32, _L) * _L

    def chunk_body(g, _):
        base = wid * _BPW + g * _CHUNK
        pltpu.sync_copy(t_hbm.at[pl.ds(base, _CHUNK)], t_idx)
        pltpu.sync_copy(c_hbm.at[pl.ds(base, _CHUNK)], c_idx)
        pltpu.sync_copy(negt_hbm.at[:, pl.ds(base, _CHUNK)], n_idx)
        cps = [pltpu.async_copy(u_hbm.at[t_idx], u_rows, sem),
               pltpu.async_copy(v_hbm.at[c_idx], c_rows, sem)]
        cps += [pltpu.async_copy(v_hbm.at[n_idx.at[n]], n_rows.at[n], sem)
                for n in range(_N_NEG)]
        for cp in cps:
            cp.wait()

        def group_body(gg, _):
            for j in range(_L):
                b = gg * _L + j
                u = [u_rows[b, pl.ds(k * _L, _L)] for k in range(_KD)]
                pacc = None
                nacc = None
                for k in range(_KD):
                    ck = c_rows[b, pl.ds(k * _L, _L)]
                    pk = u[k] * ck
                    pacc = pk if pacc is None else pacc + pk
                    sk = n_rows[0, b, pl.ds(k * _L, _L)]
                    for n in range(1, _N_NEG):
                        sk = sk + n_rows[n, b, pl.ds(k * _L, _L)]
                    nk = u[k] * sk
                    nacc = nk if nacc is None else nacc + nk
                plsc.store_scatter(scr_p, [iota16 + j], pacc)
                plsc.store_scatter(scr_n, [iota16 + j], nacc)
            tp = scr_p[pl.ds(0, _L)]
            tn = scr_n[pl.ds(0, _L)]
            for l in range(1, _L):
                tp = tp + scr_p[pl.ds(l * _L, _L)]
                tn = tn + scr_n[pl.ds(l * _L, _L)]
            pos_v[pl.ds(gg * _L, _L)] = tp
            neg_v[pl.ds(gg * _L, _L)] = tn
            return 0

        lax.fori_loop(0, _CHUNK // _L, group_body, 0)
        pltpu.sync_copy(pos_v, pos_hbm.at[pl.ds(base, _CHUNK)])
        pltpu.sync_copy(neg_v, neg_hbm.at[pl.ds(base, _CHUNK)])
        return 0

    lax.fori_loop(0, _NCHUNK, chunk_body, 0)


@functools.partial(jax.jit, static_argnums=())
def _sc_dots(t, c, neg_t, u, v):
    mesh = plsc.VectorSubcoreMesh(core_axis_name="c", subcore_axis_name="s",
                                  num_cores=_NC, num_subcores=_NS)
    f = pl.kernel(
        _sc_dots_body,
        out_type=(jax.ShapeDtypeStruct((_BATCH,), jnp.float32),
                  jax.ShapeDtypeStruct((_BATCH,), jnp.float32)),
        mesh=mesh,
        scratch_types=[
            pltpu.VMEM((_CHUNK,), jnp.int32),
            pltpu.VMEM((_CHUNK,), jnp.int32),
            pltpu.VMEM((_N_NEG, _CHUNK), jnp.int32),
            pltpu.VMEM((_CHUNK, _DIM), jnp.float32),
            pltpu.VMEM((_CHUNK, _DIM), jnp.float32),
            pltpu.VMEM((_N_NEG, _CHUNK, _DIM), jnp.float32),
            pltpu.VMEM((_CHUNK,), jnp.float32),
            pltpu.VMEM((_CHUNK,), jnp.float32),
            pltpu.VMEM((_L * _L,), jnp.float32),
            pltpu.VMEM((_L * _L,), jnp.float32),
            pltpu.SemaphoreType.DMA,
        ],
        compiler_params=pltpu.CompilerParams(needs_layout_passes=False,
                                             use_tc_tiling_on_sc=False),
    )
    return f(t, c, neg_t, u, v)


def _tc_loss_body(pos_ref, neg_ref, out_ref):
    p = pos_ref[...]
    n = neg_ref[...]
    lsp = jnp.minimum(p, 0.0) - jnp.log1p(jnp.exp(-jnp.abs(p)))
    lsn = jnp.minimum(-n, 0.0) - jnp.log1p(jnp.exp(-jnp.abs(n)))
    out_ref[0, 0] = -(jnp.sum(lsp) + jnp.sum(lsn))


def _tc_loss(pos2d, neg2d):
    return pl.pallas_call(
        _tc_loss_body,
        out_shape=jax.ShapeDtypeStruct((1, 1), jnp.float32),
        out_specs=pl.BlockSpec(memory_space=pltpu.SMEM),
    )(pos2d, neg2d)


def kernel(target_word, context_word, negative, u_weight, v_weight):
    neg_t = negative.T
    pos_d, neg_d = _sc_dots(target_word, context_word, neg_t,
                            u_weight, v_weight)
    loss = _tc_loss(pos_d.reshape(128, 128), neg_d.reshape(128, 128))
    return loss[0, 0]


# speedup vs baseline: 2.6230x; 1.5081x over previous
import functools

import jax
import jax.numpy as jnp
from jax import lax
from jax.experimental import pallas as pl
from jax.experimental.pallas import tpu as pltpu
from jax.experimental.pallas import tpu_sc as plsc

_VOCAB = 1_000_000
_DIM = 64
_BATCH = 16384
_N_NEG = 5

_NC, _NS, _L = 2, 16, 16
_NW = _NC * _NS
_BPW = _BATCH // _NW
_CHUNK = 64
_NCHUNK = _BPW // _CHUNK
_KD = _DIM // _L


def _sc_dots_body(t_hbm, c_hbm, negt_hbm, u_hbm, v_hbm,
                  pos_hbm, neg_hbm,
                  idx_v,
                  u_a, u_b, c_a, c_b, n_a, n_b,
                  pos_v, neg_v, scr_p, scr_n, sem0, sem1):
    wid = lax.axis_index("s") * _NC + lax.axis_index("c")
    iota16 = lax.iota(jnp.int32, _L) * _L
    sems = (sem0, sem1)
    u_bufs = (u_a, u_b)
    c_bufs = (c_a, c_b)
    n_bufs = (n_a, n_b)

    def enqueue(g, p):
        base = wid * _BPW + g * _CHUNK
        sem = sems[p]
        u_rows = u_bufs[p]
        c_rows = c_bufs[p]
        n_rows = n_bufs[p]
        pltpu.sync_copy(t_hbm.at[pl.ds(base, _CHUNK)],
                        idx_v.at[pl.ds(0, _CHUNK)])
        pltpu.sync_copy(c_hbm.at[pl.ds(base, _CHUNK)],
                        idx_v.at[pl.ds(_CHUNK, _CHUNK)])
        for n in range(_N_NEG):
            pltpu.sync_copy(negt_hbm.at[pl.ds(n * _BATCH + base, _CHUNK)],
                            idx_v.at[pl.ds((2 + n) * _CHUNK, _CHUNK)])
        def fetch(q, _):
            tv = idx_v[pl.ds(q * _L, _L)]
            cv = idx_v[pl.ds(_CHUNK + q * _L, _L)]
            nv = [idx_v[pl.ds((2 + n) * _CHUNK + q * _L, _L)]
                  for n in range(_N_NEG)]
            for j in range(_L):
                i = q * _L + j
                pltpu.async_copy(u_hbm.at[pl.ds(tv[j], 1)],
                                 u_rows.at[pl.ds(i, 1)], sem)
                pltpu.async_copy(v_hbm.at[pl.ds(cv[j], 1)],
                                 c_rows.at[pl.ds(i, 1)], sem)
                for n in range(_N_NEG):
                    pltpu.async_copy(v_hbm.at[pl.ds(nv[n][j], 1)],
                                     n_rows.at[pl.ds(n * _CHUNK + i, 1)],
                                     sem)
            return 0

        lax.fori_loop(0, _CHUNK // _L, fetch, 0)

    def drain(p):
        sem = sems[p]
        pltpu.make_async_copy(u_hbm.at[pl.ds(0, _CHUNK)],
                              u_bufs[p], sem).wait()
        pltpu.make_async_copy(u_hbm.at[pl.ds(0, _CHUNK)],
                              c_bufs[p], sem).wait()
        pltpu.make_async_copy(u_hbm.at[pl.ds(0, _N_NEG * _CHUNK)],
                              n_bufs[p], sem).wait()

    def compute(g, p):
        u_rows = u_bufs[p]
        c_rows = c_bufs[p]
        n_rows = n_bufs[p]

        def group_body(gg, _):
            for j in range(_L):
                b = gg * _L + j
                pacc = None
                nacc = None
                for k in range(_KD):
                    uk = u_rows[b, pl.ds(k * _L, _L)]
                    ck = c_rows[b, pl.ds(k * _L, _L)]
                    pk = uk * ck
                    pacc = pk if pacc is None else pacc + pk
                    sk = n_rows[b, pl.ds(k * _L, _L)]
                    for n in range(1, _N_NEG):
                        sk = sk + n_rows[n * _CHUNK + b, pl.ds(k * _L, _L)]
                    nk = uk * sk
                    nacc = nk if nacc is None else nacc + nk
                plsc.store_scatter(scr_p, [iota16 + j], pacc)
                plsc.store_scatter(scr_n, [iota16 + j], nacc)
            tp = scr_p[pl.ds(0, _L)]
            tn = scr_n[pl.ds(0, _L)]
            for l in range(1, _L):
                tp = tp + scr_p[pl.ds(l * _L, _L)]
                tn = tn + scr_n[pl.ds(l * _L, _L)]
            off = g * _CHUNK
            pos_v[pl.ds(off + gg * _L, _L)] = tp
            neg_v[pl.ds(off + gg * _L, _L)] = tn
            return 0

        lax.fori_loop(0, _CHUNK // _L, group_body, 0)

    enqueue(0, 0)

    def pair_body(m, _):
        g0 = 2 * m
        enqueue(g0 + 1, 1)
        drain(0)
        compute(g0, 0)

        @pl.when(m < _NCHUNK // 2 - 1)
        def _():
            enqueue(g0 + 2, 0)

        drain(1)
        compute(g0 + 1, 1)
        return 0

    lax.fori_loop(0, _NCHUNK // 2, pair_body, 0)
    pltpu.sync_copy(pos_v, pos_hbm.at[pl.ds(wid * _BPW, _BPW)])
    pltpu.sync_copy(neg_v, neg_hbm.at[pl.ds(wid * _BPW, _BPW)])


@functools.partial(jax.jit, static_argnums=())
def _sc_dots(t, c, neg_t, u, v):
    mesh = plsc.VectorSubcoreMesh(core_axis_name="c", subcore_axis_name="s",
                                  num_cores=_NC, num_subcores=_NS)
    f = pl.kernel(
        _sc_dots_body,
        out_type=(jax.ShapeDtypeStruct((_BATCH,), jnp.float32),
                  jax.ShapeDtypeStruct((_BATCH,), jnp.float32)),
        mesh=mesh,
        scratch_types=[
            pltpu.VMEM(((2 + _N_NEG) * _CHUNK,), jnp.int32),
            pltpu.VMEM((_CHUNK, _DIM), jnp.float32),
            pltpu.VMEM((_CHUNK, _DIM), jnp.float32),
            pltpu.VMEM((_CHUNK, _DIM), jnp.float32),
            pltpu.VMEM((_CHUNK, _DIM), jnp.float32),
            pltpu.VMEM((_N_NEG * _CHUNK, _DIM), jnp.float32),
            pltpu.VMEM((_N_NEG * _CHUNK, _DIM), jnp.float32),
            pltpu.VMEM((_BPW,), jnp.float32),
            pltpu.VMEM((_BPW,), jnp.float32),
            pltpu.VMEM((_L * _L,), jnp.float32),
            pltpu.VMEM((_L * _L,), jnp.float32),
            pltpu.SemaphoreType.DMA,
            pltpu.SemaphoreType.DMA,
        ],
        compiler_params=pltpu.CompilerParams(needs_layout_passes=False,
                                             use_tc_tiling_on_sc=True),
    )
    return f(t, c, neg_t, u, v)


def _tc_loss_body(pos_ref, neg_ref, out_ref):
    p = pos_ref[...]
    n = neg_ref[...]
    lsp = jnp.minimum(p, 0.0) - jnp.log1p(jnp.exp(-jnp.abs(p)))
    lsn = jnp.minimum(-n, 0.0) - jnp.log1p(jnp.exp(-jnp.abs(n)))
    out_ref[0, 0] = -(jnp.sum(lsp) + jnp.sum(lsn))


def _tc_loss(pos2d, neg2d):
    return pl.pallas_call(
        _tc_loss_body,
        out_shape=jax.ShapeDtypeStruct((1, 1), jnp.float32),
        out_specs=pl.BlockSpec(memory_space=pltpu.SMEM),
    )(pos2d, neg2d)


def kernel(target_word, context_word, negative, u_weight, v_weight):
    neg_t = negative.T.reshape(-1)
    pos_d, neg_d = _sc_dots(target_word, context_word, neg_t,
                            u_weight, v_weight)
    loss = _tc_loss(pos_d.reshape(128, 128), neg_d.reshape(128, 128))
    return loss[0, 0]


# speedup vs baseline: 3.8697x; 1.4753x over previous
import functools

import jax
import jax.numpy as jnp
from jax import lax
from jax.experimental import pallas as pl
from jax.experimental.pallas import tpu as pltpu
from jax.experimental.pallas import tpu_sc as plsc

_VOCAB = 1_000_000
_DIM = 64
_BATCH = 16384
_N_NEG = 5

_NC, _NS, _L = 2, 16, 16
_NW = _NC * _NS
_BPW = _BATCH // _NW
_CHUNK = 64
_NCHUNK = _BPW // _CHUNK
_KD = _DIM // _L


def _sc_dots_body(t_hbm, c_hbm, negt_hbm, u_hbm, v_hbm,
                  pos_hbm, neg_hbm,
                  idx_v,
                  u_a, u_b, c_a, c_b, n_a, n_b,
                  pos_v, neg_v, scr_p, scr_n, sem0, sem1):
    wid = lax.axis_index("s") * _NC + lax.axis_index("c")
    iota16 = lax.iota(jnp.int32, _L) * _L
    sems = (sem0, sem1)
    u_bufs = (u_a, u_b)
    c_bufs = (c_a, c_b)
    n_bufs = (n_a, n_b)

    def enqueue(g, p):
        base = wid * _BPW + g * _CHUNK
        sem = sems[p]
        u_rows = u_bufs[p]
        c_rows = c_bufs[p]
        n_rows = n_bufs[p]
        pltpu.sync_copy(t_hbm.at[pl.ds(base, _CHUNK)],
                        idx_v.at[pl.ds(0, _CHUNK)])
        pltpu.sync_copy(c_hbm.at[pl.ds(base, _CHUNK)],
                        idx_v.at[pl.ds(_CHUNK, _CHUNK)])
        for n in range(_N_NEG):
            pltpu.sync_copy(negt_hbm.at[pl.ds(n * _BATCH + base, _CHUNK)],
                            idx_v.at[pl.ds((2 + n) * _CHUNK, _CHUNK)])

        def fetch(q, _):
            tv = idx_v[pl.ds(q * _L, _L)]
            cv = idx_v[pl.ds(_CHUNK + q * _L, _L)]
            nv = [idx_v[pl.ds((2 + n) * _CHUNK + q * _L, _L)]
                  for n in range(_N_NEG)]
            for j in range(_L):
                i = q * _L + j
                s = tv[j]
                pltpu.async_copy(u_hbm.at[s >> 3, pl.ds(s & 7, 1)],
                                 u_rows.at[i >> 3, pl.ds(i & 7, 1)], sem)
                s = cv[j]
                pltpu.async_copy(v_hbm.at[s >> 3, pl.ds(s & 7, 1)],
                                 c_rows.at[i >> 3, pl.ds(i & 7, 1)], sem)
                for n in range(_N_NEG):
                    s = nv[n][j]
                    i2 = n * _CHUNK + i
                    pltpu.async_copy(v_hbm.at[s >> 3, pl.ds(s & 7, 1)],
                                     n_rows.at[i2 >> 3, pl.ds(i2 & 7, 1)],
                                     sem)
            return 0

        lax.fori_loop(0, _CHUNK // _L, fetch, 0)

    def drain(p):
        sem = sems[p]
        pltpu.make_async_copy(u_hbm.at[pl.ds(0, _CHUNK // 8)],
                              u_bufs[p], sem).wait()
        pltpu.make_async_copy(u_hbm.at[pl.ds(0, _CHUNK // 8)],
                              c_bufs[p], sem).wait()
        pltpu.make_async_copy(u_hbm.at[pl.ds(0, _N_NEG * _CHUNK // 8)],
                              n_bufs[p], sem).wait()

    def compute(g, p):
        u_rows = u_bufs[p]
        c_rows = c_bufs[p]
        n_rows = n_bufs[p]

        def group_body(gg, _):
            for j in range(_L):
                b = gg * _L + j
                bh, bl = b >> 3, b & 7
                pacc = None
                nacc = None
                for k in range(_KD):
                    uk = u_rows[bh, bl, pl.ds(k * _L, _L)]
                    ck = c_rows[bh, bl, pl.ds(k * _L, _L)]
                    pk = uk * ck
                    pacc = pk if pacc is None else pacc + pk
                    sk = n_rows[bh, bl, pl.ds(k * _L, _L)]
                    for n in range(1, _N_NEG):
                        b2 = n * _CHUNK + b
                        sk = sk + n_rows[b2 >> 3, b2 & 7, pl.ds(k * _L, _L)]
                    nk = uk * sk
                    nacc = nk if nacc is None else nacc + nk
                plsc.store_scatter(scr_p, [iota16 + j], pacc)
                plsc.store_scatter(scr_n, [iota16 + j], nacc)
            tp = scr_p[pl.ds(0, _L)]
            tn = scr_n[pl.ds(0, _L)]
            for l in range(1, _L):
                tp = tp + scr_p[pl.ds(l * _L, _L)]
                tn = tn + scr_n[pl.ds(l * _L, _L)]
            off = g * _CHUNK
            pos_v[pl.ds(off + gg * _L, _L)] = tp
            neg_v[pl.ds(off + gg * _L, _L)] = tn
            return 0

        lax.fori_loop(0, _CHUNK // _L, group_body, 0)

    enqueue(0, 0)

    def pair_body(m, _):
        g0 = 2 * m
        enqueue(g0 + 1, 1)
        drain(0)
        compute(g0, 0)

        @pl.when(m < _NCHUNK // 2 - 1)
        def _():
            enqueue(g0 + 2, 0)

        drain(1)
        compute(g0 + 1, 1)
        return 0

    lax.fori_loop(0, _NCHUNK // 2, pair_body, 0)
    pltpu.sync_copy(pos_v, pos_hbm.at[pl.ds(wid * _BPW, _BPW)])
    pltpu.sync_copy(neg_v, neg_hbm.at[pl.ds(wid * _BPW, _BPW)])


@functools.partial(jax.jit, static_argnums=())
def _sc_dots(t, c, neg_t, u3, v3):
    mesh = plsc.VectorSubcoreMesh(core_axis_name="c", subcore_axis_name="s",
                                  num_cores=_NC, num_subcores=_NS)
    f = pl.kernel(
        _sc_dots_body,
        out_type=(jax.ShapeDtypeStruct((_BATCH,), jnp.float32),
                  jax.ShapeDtypeStruct((_BATCH,), jnp.float32)),
        mesh=mesh,
        scratch_types=[
            pltpu.VMEM(((2 + _N_NEG) * _CHUNK,), jnp.int32),
            pltpu.VMEM((_CHUNK // 8, 8, _DIM), jnp.float32),
            pltpu.VMEM((_CHUNK // 8, 8, _DIM), jnp.float32),
            pltpu.VMEM((_CHUNK // 8, 8, _DIM), jnp.float32),
            pltpu.VMEM((_CHUNK // 8, 8, _DIM), jnp.float32),
            pltpu.VMEM((_N_NEG * _CHUNK // 8, 8, _DIM), jnp.float32),
            pltpu.VMEM((_N_NEG * _CHUNK // 8, 8, _DIM), jnp.float32),
            pltpu.VMEM((_BPW,), jnp.float32),
            pltpu.VMEM((_BPW,), jnp.float32),
            pltpu.VMEM((_L * _L,), jnp.float32),
            pltpu.VMEM((_L * _L,), jnp.float32),
            pltpu.SemaphoreType.DMA,
            pltpu.SemaphoreType.DMA,
        ],
        compiler_params=pltpu.CompilerParams(needs_layout_passes=False,
                                             use_tc_tiling_on_sc=True),
    )
    return f(t, c, neg_t, u3, v3)


def _tc_loss_body(pos_ref, neg_ref, out_ref):
    p = pos_ref[...]
    n = neg_ref[...]
    lsp = jnp.minimum(p, 0.0) - jnp.log1p(jnp.exp(-jnp.abs(p)))
    lsn = jnp.minimum(-n, 0.0) - jnp.log1p(jnp.exp(-jnp.abs(n)))
    out_ref[0, 0] = -(jnp.sum(lsp) + jnp.sum(lsn))


def _tc_loss(pos2d, neg2d):
    return pl.pallas_call(
        _tc_loss_body,
        out_shape=jax.ShapeDtypeStruct((1, 1), jnp.float32),
        out_specs=pl.BlockSpec(memory_space=pltpu.SMEM),
    )(pos2d, neg2d)


def kernel(target_word, context_word, negative, u_weight, v_weight):
    neg_t = negative.T.reshape(-1)
    u3 = u_weight.reshape(_VOCAB // 8, 8, _DIM)
    v3 = v_weight.reshape(_VOCAB // 8, 8, _DIM)
    pos_d, neg_d = _sc_dots(target_word, context_word, neg_t, u3, v3)
    loss = _tc_loss(pos_d.reshape(128, 128), neg_d.reshape(128, 128))
    return loss[0, 0]
